# baseline (speedup 1.0000x reference)
import functools

import jax
import jax.numpy as jnp
from jax import lax
from jax.experimental import pallas as pl
from jax.experimental.pallas import tpu as pltpu
from jax.experimental.pallas import tpu_sc as plsc

N = 1_000_000
D = 16
NC = 2
NS = 16
NW = NC * NS
PACK = 128 // D
NP = N // PACK
CB = 200
CK = CB * PACK
G = CK // 16
NCHUNK = NP // CB
TT = ((-(-NCHUNK // NW) + 1) // 2) * 2

_mesh = plsc.VectorSubcoreMesh(core_axis_name="c", subcore_axis_name="s")


@functools.partial(
    pl.kernel,
    out_type=(
        jax.ShapeDtypeStruct((NW * 16,), jnp.float32),
        jax.ShapeDtypeStruct((NW * 16,), jnp.int32),
    ),
    mesh=_mesh,
    compiler_params=pltpu.CompilerParams(needs_layout_passes=False),
    scratch_types=(
        pltpu.VMEM((D,), jnp.float32),
        pltpu.VMEM((CB, 128), jnp.float32),
        pltpu.VMEM((CB, 128), jnp.float32),
        pltpu.VMEM((16,), jnp.float32),
        pltpu.VMEM((16,), jnp.int32),
        pltpu.SemaphoreType.DMA,
        pltpu.SemaphoreType.DMA,
    ),
)
def _nn_scan(query_hbm, keys_hbm, score_out, idx_out,
             qv, buf0, buf1, sbest, ibest, sem0, sem1):
    cid = lax.axis_index("c")
    sid = lax.axis_index("s")
    wid = sid * NC + cid

    pltpu.sync_copy(query_hbm, qv)
    q = qv[...]
    qb = [jnp.broadcast_to(q[d], (16,)) for d in range(D)]
    iota = lax.iota(jnp.int32, 16)
    hi8 = lax.shift_right_logical(iota, 3)
    lanepat = (iota & 7) * D
    lanes = [lanepat + d for d in range(D)]

    def start(t, buf, sem):
        c = wid + NW * t
        row = pl.multiple_of(jnp.where(c < NCHUNK, c * CB, 0), 8)
        return pltpu.async_copy(keys_hbm.at[pl.ds(row, CB)], buf, sem)

    def compute(t, buf, bs, bi):
        c = wid + NW * t
        kbase = c * CK
        cvalid = c < NCHUNK

        def group_body(g, carry):
            bs, bi = carry
            rowv = hi8 + g * 2
            acc = jnp.zeros((16,), jnp.float32)
            for d in range(D):
                col = plsc.load_gather(buf, [rowv, lanes[d]])
                t_ = col - qb[d]
                acc = acc + t_ * t_
            m = cvalid & (acc < bs)
            bs = jnp.where(m, acc, bs)
            bi = jnp.where(m, kbase + g * 16 + iota, bi)
            return bs, bi

        return lax.fori_loop(0, G, group_body, (bs, bi))

    start(0, buf0, sem0)
    start(1, buf1, sem1)
    bs0 = jnp.full((16,), jnp.inf, jnp.float32)
    bi0 = jnp.zeros((16,), jnp.int32)

    def round_body(tt, carry):
        bs, bi = carry
        t0 = 2 * tt
        pltpu.make_async_copy(keys_hbm.at[pl.ds(0, CB)], buf0, sem0).wait()
        bs, bi = compute(t0, buf0, bs, bi)

        @pl.when(t0 + 2 < TT)
        def _():
            start(t0 + 2, buf0, sem0)

        pltpu.make_async_copy(keys_hbm.at[pl.ds(0, CB)], buf1, sem1).wait()
        bs, bi = compute(t0 + 1, buf1, bs, bi)

        @pl.when(t0 + 3 < TT)
        def _():
            start(t0 + 3, buf1, sem1)

        return bs, bi

    bs, bi = lax.fori_loop(0, TT // 2, round_body, (bs0, bi0))

    sbest[...] = bs
    ibest[...] = bi
    pltpu.sync_copy(sbest, score_out.at[pl.ds(wid * 16, 16)])
    pltpu.sync_copy(ibest, idx_out.at[pl.ds(wid * 16, 16)])


@functools.partial(
    pl.kernel,
    out_type=jax.ShapeDtypeStruct((16,), jnp.float32),
    mesh=_mesh,
    compiler_params=pltpu.CompilerParams(needs_layout_passes=False),
    scratch_types=(
        pltpu.VMEM((NW * 16,), jnp.float32),
        pltpu.VMEM((NW * 16,), jnp.int32),
        pltpu.VMEM((D,), jnp.float32),
        pltpu.VMEM((8, D), jnp.float32),
        pltpu.VMEM((16,), jnp.float32),
        pltpu.SemaphoreType.DMA,
    ),
)
def _nn_finish(query_hbm, values_hbm, score_hbm, idx_hbm, out_hbm,
               sbuf, ibuf, qv, vblk, ob, sem):
    cid = lax.axis_index("c")
    sid = lax.axis_index("s")

    @pl.when((cid == 0) & (sid == 0))
    def _():
        pltpu.sync_copy(score_hbm, sbuf)
        pltpu.sync_copy(idx_hbm, ibuf)
        pltpu.sync_copy(query_hbm, qv)
        bs = sbuf[pl.ds(0, 16)]
        bi = ibuf[pl.ds(0, 16)]
        for r in range(1, NW):
            s = sbuf[pl.ds(r * 16, 16)]
            i = ibuf[pl.ds(r * 16, 16)]
            m = s < bs
            bs = jnp.where(m, s, bs)
            bi = jnp.where(m, i, bi)
        minv = jnp.min(bs)
        rid = jnp.min(jnp.where(bs == minv, bi, jnp.int32(2**31 - 1)))
        base = pl.multiple_of((rid // 8) * 8, 8)
        sub = jnp.broadcast_to(rid - base, (16,))
        pltpu.async_copy(values_hbm.at[pl.ds(base, 8)], vblk, sem).wait()
        row = jnp.zeros((16,), jnp.float32)
        for r in range(8):
            row = jnp.where(sub == r, vblk[r], row)
        p = row * qv[...]
        ob[...] = jnp.broadcast_to(jnp.sum(p), (16,))
        pltpu.sync_copy(ob, out_hbm)


def kernel(query, keys, values):
    keys2 = jnp.reshape(keys, (NP, PACK * D))
    scores, ids = _nn_scan(query, keys2)
    out16 = _nn_finish(query, values, scores, ids)
    return out16[:1]


# speedup vs baseline: 1.1162x; 1.1162x over previous
import functools

import jax
import jax.numpy as jnp
from jax import lax
from jax.experimental import pallas as pl
from jax.experimental.pallas import tpu as pltpu
from jax.experimental.pallas import tpu_sc as plsc

N = 1_000_000
D = 16
NC = 2
NS = 16
NW = NC * NS
CK = 400
G = CK // 16
NCHUNK = N // CK
TT = ((-(-NCHUNK // NW) + 1) // 2) * 2

_mesh = plsc.VectorSubcoreMesh(core_axis_name="c", subcore_axis_name="s")


@functools.partial(
    pl.kernel,
    out_type=(
        jax.ShapeDtypeStruct((NW * 16,), jnp.float32),
        jax.ShapeDtypeStruct((NW * 16,), jnp.int32),
    ),
    mesh=_mesh,
    compiler_params=pltpu.CompilerParams(
        needs_layout_passes=False, use_tc_tiling_on_sc=True),
    scratch_types=(
        pltpu.VMEM((D,), jnp.float32),
        pltpu.VMEM((CK, D), jnp.float32),
        pltpu.VMEM((CK, D), jnp.float32),
        pltpu.VMEM((16,), jnp.float32),
        pltpu.VMEM((16,), jnp.int32),
        pltpu.SemaphoreType.DMA,
        pltpu.SemaphoreType.DMA,
    ),
)
def _nn_scan(query_hbm, keys_hbm, score_out, idx_out,
             qv, buf0, buf1, sbest, ibest, sem0, sem1):
    cid = lax.axis_index("c")
    sid = lax.axis_index("s")
    wid = sid * NC + cid

    pltpu.sync_copy(query_hbm, qv)
    q = qv[...]
    qb = [jnp.broadcast_to(q[d], (16,)) for d in range(D)]
    iota = lax.iota(jnp.int32, 16)
    cols = [jnp.full((16,), d, jnp.int32) for d in range(D)]

    def start(t, buf, sem):
        c = wid + NW * t
        row = pl.multiple_of(jnp.where(c < NCHUNK, c * CK, 0), 16)
        return pltpu.async_copy(keys_hbm.at[pl.ds(row, CK)], buf, sem)

    def compute(t, buf, bs, bi):
        c = wid + NW * t
        kbase = c * CK
        cvalid = c < NCHUNK

        def group_body(g, carry):
            bs, bi = carry
            rowv = iota + g * 16
            acc = jnp.zeros((16,), jnp.float32)
            for d in range(D):
                col = plsc.load_gather(buf, [rowv, cols[d]])
                t_ = col - qb[d]
                acc = acc + t_ * t_
            m = cvalid & (acc < bs)
            bs = jnp.where(m, acc, bs)
            bi = jnp.where(m, kbase + g * 16 + iota, bi)
            return bs, bi

        return lax.fori_loop(0, G, group_body, (bs, bi))

    start(0, buf0, sem0)
    start(1, buf1, sem1)
    bs0 = jnp.full((16,), jnp.inf, jnp.float32)
    bi0 = jnp.zeros((16,), jnp.int32)

    def round_body(tt, carry):
        bs, bi = carry
        t0 = 2 * tt
        pltpu.make_async_copy(keys_hbm.at[pl.ds(0, CK)], buf0, sem0).wait()
        bs, bi = compute(t0, buf0, bs, bi)

        @pl.when(t0 + 2 < TT)
        def _():
            start(t0 + 2, buf0, sem0)

        pltpu.make_async_copy(keys_hbm.at[pl.ds(0, CK)], buf1, sem1).wait()
        bs, bi = compute(t0 + 1, buf1, bs, bi)

        @pl.when(t0 + 3 < TT)
        def _():
            start(t0 + 3, buf1, sem1)

        return bs, bi

    bs, bi = lax.fori_loop(0, TT // 2, round_body, (bs0, bi0))

    sbest[...] = bs
    ibest[...] = bi
    pltpu.sync_copy(sbest, score_out.at[pl.ds(wid * 16, 16)])
    pltpu.sync_copy(ibest, idx_out.at[pl.ds(wid * 16, 16)])


@functools.partial(
    pl.kernel,
    out_type=jax.ShapeDtypeStruct((16,), jnp.float32),
    mesh=_mesh,
    compiler_params=pltpu.CompilerParams(needs_layout_passes=False),
    scratch_types=(
        pltpu.VMEM((NW * 16,), jnp.float32),
        pltpu.VMEM((NW * 16,), jnp.int32),
        pltpu.VMEM((D,), jnp.float32),
        pltpu.VMEM((8, D), jnp.float32),
        pltpu.VMEM((16,), jnp.float32),
        pltpu.SemaphoreType.DMA,
    ),
)
def _nn_finish(query_hbm, values_hbm, score_hbm, idx_hbm, out_hbm,
               sbuf, ibuf, qv, vblk, ob, sem):
    cid = lax.axis_index("c")
    sid = lax.axis_index("s")

    @pl.when((cid == 0) & (sid == 0))
    def _():
        pltpu.sync_copy(score_hbm, sbuf)
        pltpu.sync_copy(idx_hbm, ibuf)
        pltpu.sync_copy(query_hbm, qv)
        bs = sbuf[pl.ds(0, 16)]
        bi = ibuf[pl.ds(0, 16)]
        for r in range(1, NW):
            s = sbuf[pl.ds(r * 16, 16)]
            i = ibuf[pl.ds(r * 16, 16)]
            m = s < bs
            bs = jnp.where(m, s, bs)
            bi = jnp.where(m, i, bi)
        minv = jnp.min(bs)
        rid = jnp.min(jnp.where(bs == minv, bi, jnp.int32(2**31 - 1)))
        base = pl.multiple_of((rid // 8) * 8, 8)
        sub = jnp.broadcast_to(rid - base, (16,))
        pltpu.async_copy(values_hbm.at[pl.ds(base, 8)], vblk, sem).wait()
        row = jnp.zeros((16,), jnp.float32)
        for r in range(8):
            row = jnp.where(sub == r, vblk[r], row)
        p = row * qv[...]
        ob[...] = jnp.broadcast_to(jnp.sum(p), (16,))
        pltpu.sync_copy(ob, out_hbm)


def kernel(query, keys, values):
    scores, ids = _nn_scan(query, keys)
    out16 = _nn_finish(query, values, scores, ids)
    return out16[:1]
